# baseline (speedup 1.0000x reference)
import functools

import jax
import jax.numpy as jnp
from jax import lax
from jax.experimental import pallas as pl
from jax.experimental.pallas import tpu as pltpu
from jax.experimental.pallas import tpu_sc as plsc

D_MODEL = 1024
SEQ_LEN = 4096

_NC = 2
_NS = 16
_NW = _NC * _NS
_B_PER_W = SEQ_LEN // _NW
_CHUNK = 32
_NCHUNK = _B_PER_W // _CHUNK
_NBUF = 3


def _embed_body(table_hbm, idx_hbm, out_hbm, idx_v,
                b0, b1, b2, sg0, sg1, sg2, ss0, ss1, ss2):
    bufs = (b0, b1, b2)
    sgs = (sg0, sg1, sg2)
    sss = (ss0, ss1, ss2)
    wid = lax.axis_index("s") * _NC + lax.axis_index("c")
    base = wid * _B_PER_W
    for c in range(_NCHUNK):
        pltpu.sync_copy(
            idx_hbm.at[pl.ds(base + c * _CHUNK, _CHUNK)], idx_v.at[c])

    def start_g(c):
        return pltpu.async_copy(
            table_hbm.at[idx_v.at[c]], bufs[c % _NBUF], sgs[c % _NBUF])

    def start_s(c):
        return pltpu.async_copy(
            bufs[c % _NBUF],
            out_hbm.at[pl.ds(base + c * _CHUNK, _CHUNK)], sss[c % _NBUF])

    gathers = [start_g(c) for c in range(_NBUF)]
    scatters = [None] * _NCHUNK
    for c in range(_NCHUNK):
        gathers[c].wait()
        scatters[c] = start_s(c)
        if c + _NBUF < _NCHUNK:
            scatters[c].wait()
            gathers.append(start_g(c + _NBUF))
    for c in range(_NCHUNK):
        if c + _NBUF >= _NCHUNK:
            scatters[c].wait()


_embed = functools.partial(
    pl.kernel,
    mesh=plsc.VectorSubcoreMesh(core_axis_name="c", subcore_axis_name="s"),
    out_type=jax.ShapeDtypeStruct((SEQ_LEN, D_MODEL), jnp.float32),
    scratch_types=(
        [pltpu.VMEM((_NCHUNK, _CHUNK), jnp.int32)]
        + [pltpu.VMEM((_CHUNK, D_MODEL), jnp.float32) for _ in range(_NBUF)]
        + [pltpu.SemaphoreType.DMA for _ in range(2 * _NBUF)]
    ),
)(_embed_body)


@jax.jit
def kernel(tokens, W_E):
    return _embed(W_E, tokens.astype(jnp.int32))


# speedup vs baseline: 1.0254x; 1.0254x over previous
import functools

import jax
import jax.numpy as jnp
from jax import lax
from jax.experimental import pallas as pl
from jax.experimental.pallas import tpu as pltpu
from jax.experimental.pallas import tpu_sc as plsc

D_MODEL = 1024
SEQ_LEN = 4096

_NC = 2
_NS = 16
_NW = _NC * _NS
_B_PER_W = SEQ_LEN // _NW
_CHUNK = 32
_NCHUNK = _B_PER_W // _CHUNK
_NBUF = 3


def _embed_body(table_hbm, idx_hbm, out_hbm, idx_v,
                b0, b1, b2, sg0, sg1, sg2, ss0, ss1, ss2):
    bufs = (b0, b1, b2)
    sgs = (sg0, sg1, sg2)
    sss = (ss0, ss1, ss2)
    wid = lax.axis_index("s") * _NC + lax.axis_index("c")
    base = wid * _B_PER_W
    idx_copies = [
        pltpu.async_copy(
            idx_hbm.at[pl.ds(base + c * _CHUNK, _CHUNK)], idx_v.at[c], sg0)
        for c in range(_NCHUNK)
    ]
    for cp in idx_copies:
        cp.wait()

    def start_g(c):
        return pltpu.async_copy(
            table_hbm.at[idx_v.at[c]], bufs[c % _NBUF], sgs[c % _NBUF])

    def start_s(c):
        return pltpu.async_copy(
            bufs[c % _NBUF],
            out_hbm.at[pl.ds(base + c * _CHUNK, _CHUNK)], sss[c % _NBUF])

    gathers = [start_g(c) for c in range(_NBUF)]
    scatters = [None] * _NCHUNK
    for c in range(_NCHUNK):
        gathers[c].wait()
        scatters[c] = start_s(c)
        if c + _NBUF < _NCHUNK:
            scatters[c].wait()
            gathers.append(start_g(c + _NBUF))
    for c in range(_NCHUNK):
        if c + _NBUF >= _NCHUNK:
            scatters[c].wait()


_embed = functools.partial(
    pl.kernel,
    mesh=plsc.VectorSubcoreMesh(core_axis_name="c", subcore_axis_name="s"),
    out_type=jax.ShapeDtypeStruct((SEQ_LEN, D_MODEL), jnp.float32),
    scratch_types=(
        [pltpu.VMEM((_NCHUNK, _CHUNK), jnp.int32)]
        + [pltpu.VMEM((_CHUNK, D_MODEL), jnp.float32) for _ in range(_NBUF)]
        + [pltpu.SemaphoreType.DMA for _ in range(2 * _NBUF)]
    ),
)(_embed_body)


@jax.jit
def kernel(tokens, W_E):
    return _embed(W_E, tokens.astype(jnp.int32))
